# baseline (speedup 1.0000x reference)
import functools

import jax
import jax.numpy as jnp
from jax import lax
from jax.experimental import pallas as pl
from jax.experimental.pallas import tpu as pltpu
from jax.experimental.pallas import tpu_sc as plsc

N = 10000
E = 320000
D = 128
DO = 64
NP = 10240
RPS = 640
CH = 128
NW = 32
NCHUNK = 84
PER_W = NCHUNK * CH
EP = NW * PER_W
NBUF = 2
ISLOT = 6

_mesh = plsc.VectorSubcoreMesh(core_axis_name="c", subcore_axis_name="s")


def _fill_rows(buf, nrows, ncols, value):
    vec = jnp.full((16,), value, jnp.float32)

    @pl.loop(0, nrows)
    def _(r):
        @pl.loop(0, ncols // 16)
        def _(j):
            buf[r, pl.ds(j * 16, 16)] = vec


def _zero_acc_share(zsrc, acc, s):
    base = s * RPS

    @pl.loop(0, RPS // CH)
    def _(k):
        pltpu.sync_copy(zsrc, acc.at[pl.ds(base + k * CH, CH)])


def _dump_acc_share(acc, out_hbm, c, s):
    pltpu.sync_copy(acc.at[pl.ds(s * RPS, RPS)],
                    out_hbm.at[c, pl.ds(s * RPS, RPS)])


@functools.partial(
    pl.kernel,
    out_type=jax.ShapeDtypeStruct((2, NP, D), jnp.float32),
    mesh=_mesh,
    scratch_types=[
        pltpu.VMEM((NCHUNK, CH), jnp.int32),
        pltpu.VMEM((CH, D), jnp.float32),
        pltpu.VMEM_SHARED((NP, D), jnp.float32),
    ],
)
def _sc_deg(dst_hbm, out_hbm, dsts, buf, acc):
    c = lax.axis_index("c")
    s = lax.axis_index("s")
    wid = c * 16 + s

    pltpu.sync_copy(dst_hbm.at[wid], dsts)
    _fill_rows(buf, CH, D, 0.0)
    _zero_acc_share(buf, acc, s)
    _fill_rows(buf, CH, D, 1.0)
    plsc.subcore_barrier()

    @pl.loop(0, NCHUNK)
    def _(i):
        pltpu.sync_copy(buf, acc.at[dsts.at[i]], add=True)

    plsc.subcore_barrier()
    _dump_acc_share(acc, out_hbm, c, s)


@functools.partial(
    pl.kernel,
    out_type=jax.ShapeDtypeStruct((2, NP, D), jnp.float32),
    mesh=_mesh,
    scratch_types=[
        pltpu.VMEM((ISLOT, 2, CH), jnp.int32),
        pltpu.VMEM((NBUF * CH, D), jnp.float32),
        pltpu.VMEM_SHARED((NP, D), jnp.float32),
        [pltpu.SemaphoreType.DMA] * ISLOT,
        [pltpu.SemaphoreType.DMA] * NBUF,
    ],
)
def _sc_edges(y_hbm, sd_hbm, out_hbm, iv, rows, acc, isems, gsems):
    c = lax.axis_index("c")
    s = lax.axis_index("s")

    _fill_rows(rows, CH, D, 0.0)
    _zero_acc_share(rows.at[pl.ds(0, CH)], acc, s)
    plsc.subcore_barrier()

    def idx_copy(start, chunk, slot):
        return pltpu.make_async_copy(sd_hbm.at[start + chunk],
                                     iv.at[slot], isems[slot])

    def gather_copy(chunk_slot, buf):
        return pltpu.make_async_copy(y_hbm.at[iv.at[chunk_slot, 0]],
                                     rows.at[pl.ds(buf * CH, CH)], gsems[buf])

    def span_pipelined(start, nchunks):
        for b in range(ISLOT):
            idx_copy(start, b, b).start()
        for b in range(NBUF):
            idx_copy(start, b, b).wait()
            gather_copy(b, b).start()

        @pl.loop(0, nchunks, step=ISLOT)
        def _(i):
            for b in range(ISLOT):
                k = i + b
                rb = b % NBUF
                gather_copy(b, rb).wait()
                pltpu.sync_copy(rows.at[pl.ds(rb * CH, CH)],
                                acc.at[iv.at[b, 1]], add=True)

                @pl.when(k + ISLOT < nchunks)
                def _():
                    idx_copy(start, k + ISLOT, b).start()

                @pl.when(k + NBUF < nchunks)
                def _():
                    sl = (b + NBUF) % ISLOT
                    idx_copy(start, k + NBUF, sl).wait()
                    gather_copy(sl, rb).start()

    wid = c * 16 + s
    span_pipelined(wid * NCHUNK, NCHUNK)

    plsc.subcore_barrier()
    _dump_acc_share(acc, out_hbm, c, s)


def _row_mask(shape):
    return lax.broadcasted_iota(jnp.int32, shape, 0) < N


def _tc_matmul_body(x_ref, w_ref, o_ref):
    o_ref[...] = jnp.dot(x_ref[...], w_ref[...],
                         preferred_element_type=jnp.float32)


def _tc_matmul(x, w):
    return pl.pallas_call(
        _tc_matmul_body,
        out_shape=jax.ShapeDtypeStruct((x.shape[0], w.shape[1]), jnp.float32),
    )(x, w)


def _tc_prep_body(degp_ref, xw_ref, d_ref, y_ref):
    degp = degp_ref[...]
    deg = degp[0, :, 0:1] + degp[1, :, 0:1] + 1.0
    d = lax.rsqrt(deg)
    d_ref[...] = d
    y = d * xw_ref[...]
    y_ref[...] = jnp.where(_row_mask(y.shape), y, 0.0)


def _tc_prep(deg_parts, xw):
    return pl.pallas_call(
        _tc_prep_body,
        out_shape=(jax.ShapeDtypeStruct((NP, 1), jnp.float32),
                   jax.ShapeDtypeStruct((NP, D), jnp.float32)),
    )(deg_parts, xw)


def _tc_mid_body(sp_ref, y_ref, d_ref, b_ref, w_ref, o_ref):
    sp = sp_ref[...]
    d = d_ref[...]
    h = sp[0] + sp[1] + y_ref[...]
    h = jnp.maximum(d * h + b_ref[...][None, :], 0.0)
    xw = jnp.dot(h, w_ref[...], preferred_element_type=jnp.float32)
    y2 = d * xw
    o_ref[...] = jnp.where(_row_mask(y2.shape), y2, 0.0)


def _tc_mid(s_parts, y, d, b, w):
    return pl.pallas_call(
        _tc_mid_body,
        out_shape=jax.ShapeDtypeStruct((NP, D), jnp.float32),
    )(s_parts, y, d, b, w)


def _tc_final_body(sp_ref, y_ref, d_ref, b_ref, w_ref, bfc_ref, o_ref):
    sp = sp_ref[...]
    h = sp[0] + sp[1] + y_ref[...]
    h = jnp.maximum(d_ref[...] * h + b_ref[...][None, :], 0.0)
    o_ref[...] = (jnp.dot(h, w_ref[...], preferred_element_type=jnp.float32)
                  + bfc_ref[...][None, :])


def _tc_final(s_parts, y, d, b, wfc, bfc):
    return pl.pallas_call(
        _tc_final_body,
        out_shape=jax.ShapeDtypeStruct((NP, DO), jnp.float32),
    )(s_parts, y, d, b, wfc, bfc)


def kernel(x, edge_index, W1, b1, W2, b2, Wfc, bfc):
    x_pad = jnp.pad(x, ((0, NP - N), (0, 0)))
    npad = EP - E
    pad_src = jnp.full((npad,), N, jnp.int32)
    pad_dst = N + 1 + jnp.arange(npad, dtype=jnp.int32) % (NP - N - 1)
    src_pad = jnp.concatenate([edge_index[0], pad_src]).reshape(EP // CH, CH)
    dst_pad = jnp.concatenate([edge_index[1], pad_dst]).reshape(EP // CH, CH)
    sd = jnp.stack([src_pad, dst_pad], axis=1)

    deg_parts = _sc_deg(dst_pad.reshape(NW, NCHUNK, CH))
    xw1 = _tc_matmul(x_pad, W1)
    d, y1 = _tc_prep(deg_parts, xw1)
    s1 = _sc_edges(y1, sd)
    y2 = _tc_mid(s1, y1, d, b1, W2)
    s2 = _sc_edges(y2, sd)
    out = _tc_final(s2, y2, d, b2, Wfc, bfc)
    return out[:N]


# speedup vs baseline: 6.8316x; 6.8316x over previous
import functools

import jax
import jax.numpy as jnp
from jax import lax
from jax.experimental import pallas as pl
from jax.experimental.pallas import tpu as pltpu
from jax.experimental.pallas import tpu_sc as plsc

N = 10000
E = 320000
D = 128
DO = 64
NP = 10240
RPS = 640
CH = 128
NW = 32
NCHUNK = 84
PER_W = NCHUNK * CH
EP = NW * PER_W
NBUF = 2
ISLOT = 6

_mesh = plsc.VectorSubcoreMesh(core_axis_name="c", subcore_axis_name="s")


def _fill_rows(buf, nrows, ncols, value):
    vec = jnp.full((16,), value, jnp.float32)

    @pl.loop(0, nrows)
    def _(r):
        @pl.loop(0, ncols // 16)
        def _(j):
            buf[r, pl.ds(j * 16, 16)] = vec


def _zero_acc_share(zsrc, acc, s):
    base = s * RPS

    @pl.loop(0, RPS // CH)
    def _(k):
        pltpu.sync_copy(zsrc, acc.at[pl.ds(base + k * CH, CH)])


def _dump_acc_share(acc, out_hbm, c, s):
    pltpu.sync_copy(acc.at[pl.ds(s * RPS, RPS)],
                    out_hbm.at[c, pl.ds(s * RPS, RPS)])


@functools.partial(
    pl.kernel,
    out_type=jax.ShapeDtypeStruct((2, NP, D), jnp.float32),
    mesh=_mesh,
    scratch_types=[
        pltpu.VMEM((NCHUNK, CH), jnp.int32),
        pltpu.VMEM((CH, D), jnp.float32),
        pltpu.VMEM_SHARED((NP, D), jnp.float32),
    ],
)
def _sc_deg(dst_hbm, out_hbm, dsts, buf, acc):
    c = lax.axis_index("c")
    s = lax.axis_index("s")
    wid = c * 16 + s

    pltpu.sync_copy(dst_hbm.at[wid], dsts)
    _fill_rows(buf, CH, D, 0.0)
    _zero_acc_share(buf, acc, s)
    _fill_rows(buf, CH, D, 1.0)
    plsc.subcore_barrier()

    @pl.loop(0, NCHUNK)
    def _(i):
        pltpu.sync_copy(buf, acc.at[dsts.at[i]], add=True)

    plsc.subcore_barrier()
    _dump_acc_share(acc, out_hbm, c, s)


@functools.partial(
    pl.kernel,
    out_type=jax.ShapeDtypeStruct((2, NP, D), jnp.float32),
    mesh=_mesh,
    scratch_types=[
        pltpu.VMEM((ISLOT, 2, CH), jnp.int32),
        pltpu.VMEM((NBUF * CH, D), jnp.float32),
        pltpu.VMEM_SHARED((NP, D), jnp.float32),
        [pltpu.SemaphoreType.DMA] * ISLOT,
        [pltpu.SemaphoreType.DMA] * NBUF,
    ],
)
def _sc_edges(y_hbm, sd_hbm, out_hbm, iv, rows, acc, isems, gsems):
    c = lax.axis_index("c")
    s = lax.axis_index("s")

    _fill_rows(rows, CH, D, 0.0)
    _zero_acc_share(rows.at[pl.ds(0, CH)], acc, s)
    plsc.subcore_barrier()

    def idx_copy(start, chunk, slot):
        return pltpu.make_async_copy(sd_hbm.at[start + chunk],
                                     iv.at[slot], isems[slot])

    def gather_copy(chunk_slot, buf):
        return pltpu.make_async_copy(y_hbm.at[iv.at[chunk_slot, 0]],
                                     rows.at[pl.ds(buf * CH, CH)], gsems[buf])

    def span_pipelined(start, nchunks):
        for b in range(ISLOT):
            idx_copy(start, b, b).start()
        for b in range(NBUF):
            idx_copy(start, b, b).wait()
            gather_copy(b, b).start()

        @pl.loop(0, nchunks, step=ISLOT)
        def _(i):
            for b in range(ISLOT):
                k = i + b
                rb = b % NBUF
                gather_copy(b, rb).wait()
                pltpu.sync_copy(rows.at[pl.ds(rb * CH, CH)],
                                acc.at[iv.at[b, 1]], add=True)

                @pl.when(k + ISLOT < nchunks)
                def _():
                    idx_copy(start, k + ISLOT, b).start()

                @pl.when(k + NBUF < nchunks)
                def _():
                    sl = (b + NBUF) % ISLOT
                    idx_copy(start, k + NBUF, sl).wait()
                    gather_copy(sl, rb).start()

    wid = c * 16 + s
    span_pipelined(wid * NCHUNK, NCHUNK)

    plsc.subcore_barrier()
    _dump_acc_share(acc, out_hbm, c, s)


def _row_mask(shape):
    return lax.broadcasted_iota(jnp.int32, shape, 0) < N


def _tc_matmul_body(x_ref, w_ref, o_ref):
    o_ref[...] = jnp.dot(x_ref[...], w_ref[...],
                         preferred_element_type=jnp.float32)


def _tc_matmul(x, w):
    return pl.pallas_call(
        _tc_matmul_body,
        out_shape=jax.ShapeDtypeStruct((x.shape[0], w.shape[1]), jnp.float32),
    )(x, w)


def _tc_prep_body(degp_ref, xw_ref, d_ref, y_ref):
    degp = degp_ref[...]
    deg = degp[0, :, 0:1] + degp[1, :, 0:1] + 1.0
    d = lax.rsqrt(deg)
    d_ref[...] = d
    y = d * xw_ref[...]
    y_ref[...] = jnp.where(_row_mask(y.shape), y, 0.0)


def _tc_prep(deg_parts, xw):
    return pl.pallas_call(
        _tc_prep_body,
        out_shape=(jax.ShapeDtypeStruct((NP, 1), jnp.float32),
                   jax.ShapeDtypeStruct((NP, D), jnp.float32)),
    )(deg_parts, xw)


def _tc_mid_body(sp_ref, y_ref, d_ref, b_ref, w_ref, o_ref):
    sp = sp_ref[...]
    d = d_ref[...]
    h = sp[0] + sp[1] + y_ref[...]
    h = jnp.maximum(d * h + b_ref[...][None, :], 0.0)
    xw = jnp.dot(h, w_ref[...], preferred_element_type=jnp.float32)
    y2 = d * xw
    o_ref[...] = jnp.where(_row_mask(y2.shape), y2, 0.0)


def _tc_mid(s_parts, y, d, b, w):
    return pl.pallas_call(
        _tc_mid_body,
        out_shape=jax.ShapeDtypeStruct((NP, D), jnp.float32),
    )(s_parts, y, d, b, w)


def _tc_final_body(sp_ref, y_ref, d_ref, b_ref, w_ref, bfc_ref, o_ref):
    sp = sp_ref[...]
    h = sp[0] + sp[1] + y_ref[...]
    h = jnp.maximum(d_ref[...] * h + b_ref[...][None, :], 0.0)
    o_ref[...] = (jnp.dot(h, w_ref[...], preferred_element_type=jnp.float32)
                  + bfc_ref[...][None, :])


def _tc_final(s_parts, y, d, b, wfc, bfc):
    return pl.pallas_call(
        _tc_final_body,
        out_shape=jax.ShapeDtypeStruct((NP, DO), jnp.float32),
    )(s_parts, y, d, b, wfc, bfc)


def kernel(x, edge_index, W1, b1, W2, b2, Wfc, bfc):
    x_pad = jnp.pad(x, ((0, NP - N), (0, 0)))
    npad = EP - E
    ar = jnp.arange(npad, dtype=jnp.int32)
    pad_src = N + 1 + (ar + 101) % (NP - N - 1)
    pad_dst = N + 1 + ar % (NP - N - 1)
    src_pad = jnp.concatenate([edge_index[0], pad_src]).reshape(EP // CH, CH)
    dst_pad = jnp.concatenate([edge_index[1], pad_dst]).reshape(EP // CH, CH)
    sd = jnp.stack([src_pad, dst_pad], axis=1)

    deg_parts = _sc_deg(dst_pad.reshape(NW, NCHUNK, CH))
    xw1 = _tc_matmul(x_pad, W1)
    d, y1 = _tc_prep(deg_parts, xw1)
    s1 = _sc_edges(y1, sd)
    y2 = _tc_mid(s1, y1, d, b1, W2)
    s2 = _sc_edges(y2, sd)
    out = _tc_final(s2, y2, d, b2, Wfc, bfc)
    return out[:N]


# speedup vs baseline: 7.0809x; 1.0365x over previous
import functools

import jax
import jax.numpy as jnp
from jax import lax
from jax.experimental import pallas as pl
from jax.experimental.pallas import tpu as pltpu
from jax.experimental.pallas import tpu_sc as plsc

N = 10000
E = 320000
D = 128
DO = 64
NP = 10240
RPS = 640
CH = 128
NW = 32
NCHUNK = 80
PER_W = NCHUNK * CH
EP = NW * PER_W
NBUF = 2
ISLOT = 8

_mesh = plsc.VectorSubcoreMesh(core_axis_name="c", subcore_axis_name="s")


def _fill_rows(buf, nrows, ncols, value):
    vec = jnp.full((16,), value, jnp.float32)

    @pl.loop(0, nrows)
    def _(r):
        @pl.loop(0, ncols // 16)
        def _(j):
            buf[r, pl.ds(j * 16, 16)] = vec


def _zero_acc_share(zsrc, acc, s):
    base = s * RPS

    @pl.loop(0, RPS // CH)
    def _(k):
        pltpu.sync_copy(zsrc, acc.at[pl.ds(base + k * CH, CH)])


def _dump_acc_share(acc, out_hbm, c, s):
    pltpu.sync_copy(acc.at[pl.ds(s * RPS, RPS)],
                    out_hbm.at[c, pl.ds(s * RPS, RPS)])


@functools.partial(
    pl.kernel,
    out_type=jax.ShapeDtypeStruct((2, NP, D), jnp.float32),
    mesh=_mesh,
    scratch_types=[
        pltpu.VMEM((NCHUNK, CH), jnp.int32),
        pltpu.VMEM((CH, D), jnp.float32),
        pltpu.VMEM_SHARED((NP, D), jnp.float32),
    ],
)
def _sc_deg(dst_hbm, out_hbm, dsts, buf, acc):
    c = lax.axis_index("c")
    s = lax.axis_index("s")
    wid = c * 16 + s

    pltpu.sync_copy(dst_hbm.at[wid], dsts)
    _fill_rows(buf, CH, D, 0.0)
    _zero_acc_share(buf, acc, s)
    _fill_rows(buf, CH, D, 1.0)
    plsc.subcore_barrier()

    @pl.loop(0, NCHUNK)
    def _(i):
        pltpu.sync_copy(buf, acc.at[dsts.at[i]], add=True)

    plsc.subcore_barrier()
    _dump_acc_share(acc, out_hbm, c, s)


@functools.partial(
    pl.kernel,
    out_type=jax.ShapeDtypeStruct((2, NP, D), jnp.float32),
    mesh=_mesh,
    scratch_types=[
        pltpu.VMEM((ISLOT, 2, CH), jnp.int32),
        pltpu.VMEM((NBUF * CH, D), jnp.float32),
        pltpu.VMEM_SHARED((NP, D), jnp.float32),
        [pltpu.SemaphoreType.DMA] * ISLOT,
        [pltpu.SemaphoreType.DMA] * NBUF,
    ],
)
def _sc_edges(y_hbm, sd_hbm, out_hbm, iv, rows, acc, isems, gsems):
    c = lax.axis_index("c")
    s = lax.axis_index("s")

    _fill_rows(rows, CH, D, 0.0)
    _zero_acc_share(rows.at[pl.ds(0, CH)], acc, s)
    plsc.subcore_barrier()

    def idx_copy(start, chunk, slot):
        return pltpu.make_async_copy(sd_hbm.at[start + chunk],
                                     iv.at[slot], isems[slot])

    def gather_copy(chunk_slot, buf):
        return pltpu.make_async_copy(y_hbm.at[iv.at[chunk_slot, 0]],
                                     rows.at[pl.ds(buf * CH, CH)], gsems[buf])

    def span_pipelined(start, nchunks):
        for b in range(ISLOT):
            idx_copy(start, b, b).start()
        for b in range(NBUF):
            idx_copy(start, b, b).wait()
            gather_copy(b, b).start()

        @pl.loop(0, nchunks, step=ISLOT)
        def _(i):
            for b in range(ISLOT):
                k = i + b
                rb = b % NBUF
                gather_copy(b, rb).wait()
                pltpu.sync_copy(rows.at[pl.ds(rb * CH, CH)],
                                acc.at[iv.at[b, 1]], add=True)

                @pl.when(k + ISLOT < nchunks)
                def _():
                    idx_copy(start, k + ISLOT, b).start()

                @pl.when(k + NBUF < nchunks)
                def _():
                    sl = (b + NBUF) % ISLOT
                    idx_copy(start, k + NBUF, sl).wait()
                    gather_copy(sl, rb).start()

    wid = c * 16 + s
    span_pipelined(wid * NCHUNK, NCHUNK)

    plsc.subcore_barrier()
    _dump_acc_share(acc, out_hbm, c, s)


def _row_mask(shape):
    return lax.broadcasted_iota(jnp.int32, shape, 0) < N


def _tc_matmul_body(x_ref, w_ref, o_ref):
    o_ref[...] = jnp.dot(x_ref[...], w_ref[...],
                         preferred_element_type=jnp.float32)


def _tc_matmul(x, w):
    return pl.pallas_call(
        _tc_matmul_body,
        out_shape=jax.ShapeDtypeStruct((x.shape[0], w.shape[1]), jnp.float32),
    )(x, w)


def _tc_prep_body(degp_ref, xw_ref, d_ref, y_ref):
    degp = degp_ref[...]
    deg = degp[0, :, 0:1] + degp[1, :, 0:1] + 1.0
    d = lax.rsqrt(deg)
    d_ref[...] = d
    y = d * xw_ref[...]
    y_ref[...] = jnp.where(_row_mask(y.shape), y, 0.0)


def _tc_prep(deg_parts, xw):
    return pl.pallas_call(
        _tc_prep_body,
        out_shape=(jax.ShapeDtypeStruct((NP, 1), jnp.float32),
                   jax.ShapeDtypeStruct((NP, D), jnp.float32)),
    )(deg_parts, xw)


def _tc_mid_body(sp_ref, y_ref, d_ref, b_ref, w_ref, o_ref):
    sp = sp_ref[...]
    d = d_ref[...]
    h = sp[0] + sp[1] + y_ref[...]
    h = jnp.maximum(d * h + b_ref[...][None, :], 0.0)
    xw = jnp.dot(h, w_ref[...], preferred_element_type=jnp.float32)
    y2 = d * xw
    o_ref[...] = jnp.where(_row_mask(y2.shape), y2, 0.0)


def _tc_mid(s_parts, y, d, b, w):
    return pl.pallas_call(
        _tc_mid_body,
        out_shape=jax.ShapeDtypeStruct((NP, D), jnp.float32),
    )(s_parts, y, d, b, w)


def _tc_final_body(sp_ref, y_ref, d_ref, b_ref, w_ref, bfc_ref, o_ref):
    sp = sp_ref[...]
    h = sp[0] + sp[1] + y_ref[...]
    h = jnp.maximum(d_ref[...] * h + b_ref[...][None, :], 0.0)
    o_ref[...] = (jnp.dot(h, w_ref[...], preferred_element_type=jnp.float32)
                  + bfc_ref[...][None, :])


def _tc_final(s_parts, y, d, b, wfc, bfc):
    return pl.pallas_call(
        _tc_final_body,
        out_shape=jax.ShapeDtypeStruct((NP, DO), jnp.float32),
    )(s_parts, y, d, b, wfc, bfc)


def kernel(x, edge_index, W1, b1, W2, b2, Wfc, bfc):
    x_pad = jnp.pad(x, ((0, NP - N), (0, 0)))
    npad = EP - E
    ar = jnp.arange(npad, dtype=jnp.int32)
    pad_src = N + 1 + (ar + 101) % (NP - N - 1)
    pad_dst = N + 1 + ar % (NP - N - 1)
    src_pad = jnp.concatenate([edge_index[0], pad_src]).reshape(EP // CH, CH)
    dst_pad = jnp.concatenate([edge_index[1], pad_dst]).reshape(EP // CH, CH)
    sd = jnp.stack([src_pad, dst_pad], axis=1)

    deg_parts = _sc_deg(dst_pad.reshape(NW, NCHUNK, CH))
    xw1 = _tc_matmul(x_pad, W1)
    d, y1 = _tc_prep(deg_parts, xw1)
    s1 = _sc_edges(y1, sd)
    y2 = _tc_mid(s1, y1, d, b1, W2)
    s2 = _sc_edges(y2, sd)
    out = _tc_final(s2, y2, d, b2, Wfc, bfc)
    return out[:N]
